# speedup vs baseline: 1.0873x; 1.0873x over previous
import functools
import jax
import jax.numpy as jnp
from jax import lax
from jax.experimental import pallas as pl
from jax.experimental.pallas import tpu as pltpu
from jax.experimental.pallas import tpu_sc as plsc

_CHUNK = 128


@functools.partial(jax.jit, static_argnames=("n_chunks_per_worker",))
def _sc_gather(x2d, embedding, n_chunks_per_worker):
    info = plsc.get_sparse_core_info()
    nc = info.num_cores
    n_rows = x2d.shape[0] * _CHUNK
    d = embedding.shape[1]

    mesh = plsc.VectorSubcoreMesh(core_axis_name="c", subcore_axis_name="s")

    @functools.partial(
        pl.kernel,
        mesh=mesh,
        out_type=jax.ShapeDtypeStruct((n_rows, d), jnp.float32),
        scratch_types=[
            pltpu.VMEM((_CHUNK,), jnp.int32),
            pltpu.VMEM((_CHUNK, d), jnp.float32),
            pltpu.SemaphoreType.DMA,
        ],
    )
    def k(x_hbm, tab_hbm, out_hbm, idx_v, rows_v, sem):
        wid = lax.axis_index("s") * nc + lax.axis_index("c")

        def step(j, carry):
            r = wid * n_chunks_per_worker + j
            pltpu.sync_copy(x_hbm.at[r], idx_v)
            pltpu.async_copy(tab_hbm.at[idx_v], rows_v, sem).wait()
            pltpu.sync_copy(rows_v, out_hbm.at[pl.ds(r * _CHUNK, _CHUNK)])
            return carry

        lax.fori_loop(0, n_chunks_per_worker, step, 0)

    return k(x2d, embedding)


def kernel(X, embedding):
    b, s = X.shape
    n = b * s
    info = plsc.get_sparse_core_info()
    nw = info.num_cores * info.num_subcores
    assert n % (nw * _CHUNK) == 0
    x2d = X.reshape(n // _CHUNK, _CHUNK).astype(jnp.int32)
    out = _sc_gather(x2d, embedding, n // (nw * _CHUNK))
    return out.reshape(b, s, embedding.shape[1])


# speedup vs baseline: 1.2820x; 1.1791x over previous
import functools
import jax
import jax.numpy as jnp
from jax import lax
from jax.experimental import pallas as pl
from jax.experimental.pallas import tpu as pltpu
from jax.experimental.pallas import tpu_sc as plsc

_CHUNK = 128
_NBUF = 2


@functools.partial(jax.jit, static_argnames=("n_chunks",))
def _sc_gather(x3d, embedding, n_chunks):
    info = plsc.get_sparse_core_info()
    nc = info.num_cores
    n_rows = x3d.shape[0] * x3d.shape[1] * _CHUNK
    d = embedding.shape[1]

    mesh = plsc.VectorSubcoreMesh(core_axis_name="c", subcore_axis_name="s")

    @functools.partial(
        pl.kernel,
        mesh=mesh,
        out_type=jax.ShapeDtypeStruct((n_rows, d), jnp.float32),
        scratch_types=[
            pltpu.VMEM((n_chunks, _CHUNK), jnp.int32),
            pltpu.VMEM((_CHUNK, d), jnp.float32),
            pltpu.VMEM((_CHUNK, d), jnp.float32),
            pltpu.SemaphoreType.DMA,
            pltpu.SemaphoreType.DMA,
            pltpu.SemaphoreType.DMA,
            pltpu.SemaphoreType.DMA,
        ],
    )
    def k(x_hbm, tab_hbm, out_hbm, idx_all, rows0, rows1, gs0, gs1, ss0, ss1):
        wid = lax.axis_index("s") * nc + lax.axis_index("c")
        base = wid * n_chunks
        rows = (rows0, rows1)
        gsem = (gs0, gs1)
        ssem = (ss0, ss1)

        pltpu.sync_copy(x_hbm.at[wid], idx_all)

        for b in range(_NBUF):
            pltpu.async_copy(tab_hbm.at[idx_all.at[b]], rows[b], gsem[b])

        def outer(i, carry):
            for b in range(_NBUF):
                j = i * _NBUF + b
                pltpu.make_async_copy(
                    tab_hbm.at[idx_all.at[j]], rows[b], gsem[b]
                ).wait()
                out_slc = out_hbm.at[pl.ds((base + j) * _CHUNK, _CHUNK)]
                pltpu.async_copy(rows[b], out_slc, ssem[b])
                pltpu.make_async_copy(rows[b], out_slc, ssem[b]).wait()
                pltpu.async_copy(
                    tab_hbm.at[idx_all.at[j + _NBUF]], rows[b], gsem[b]
                )
            return carry

        lax.fori_loop(0, n_chunks // _NBUF - 1, outer, 0)

        for b in range(_NBUF):
            j = n_chunks - _NBUF + b
            pltpu.make_async_copy(
                tab_hbm.at[idx_all.at[j]], rows[b], gsem[b]
            ).wait()
            pltpu.async_copy(
                rows[b], out_hbm.at[pl.ds((base + j) * _CHUNK, _CHUNK)], ssem[b]
            )
        for b in range(_NBUF):
            j = n_chunks - _NBUF + b
            pltpu.make_async_copy(
                rows[b], out_hbm.at[pl.ds((base + j) * _CHUNK, _CHUNK)], ssem[b]
            ).wait()

    return k(x3d, embedding)


def kernel(X, embedding):
    b, s = X.shape
    n = b * s
    info = plsc.get_sparse_core_info()
    nw = info.num_cores * info.num_subcores
    n_chunks = n // (nw * _CHUNK)
    assert n == nw * n_chunks * _CHUNK and n_chunks % _NBUF == 0
    x3d = X.reshape(nw, n_chunks, _CHUNK).astype(jnp.int32)
    out = _sc_gather(x3d, embedding, n_chunks)
    return out.reshape(b, s, embedding.shape[1])


# speedup vs baseline: 2.0060x; 1.5647x over previous
import functools
import jax
import jax.numpy as jnp
from jax import lax
from jax.experimental import pallas as pl
from jax.experimental.pallas import tpu as pltpu
from jax.experimental.pallas import tpu_sc as plsc

_NBUF = 2
_RPC = 4


@functools.partial(jax.jit, static_argnames=("b", "s"))
def _sc_gather(x3d, embedding, b, s):
    info = plsc.get_sparse_core_info()
    nc = info.num_cores
    nw = nc * info.num_subcores
    d = embedding.shape[1]
    rpw = b // nw
    n_chunks = rpw // _RPC
    cidx = _RPC * s

    mesh = plsc.VectorSubcoreMesh(core_axis_name="c", subcore_axis_name="s")

    @functools.partial(
        pl.kernel,
        mesh=mesh,
        out_type=jax.ShapeDtypeStruct((b, s, d), jnp.float32),
        scratch_types=[
            pltpu.VMEM((n_chunks, cidx), jnp.int32),
            pltpu.VMEM((cidx, d), jnp.float32),
            pltpu.VMEM((cidx, d), jnp.float32),
            pltpu.SemaphoreType.DMA,
            pltpu.SemaphoreType.DMA,
            pltpu.SemaphoreType.DMA,
            pltpu.SemaphoreType.DMA,
        ],
    )
    def k(x_hbm, tab_hbm, out_hbm, idx_all, rows0, rows1, gs0, gs1, ss0, ss1):
        wid = lax.axis_index("s") * nc + lax.axis_index("c")
        base = wid * rpw
        rows = (rows0, rows1)
        gsem = (gs0, gs1)
        ssem = (ss0, ss1)

        pltpu.sync_copy(x_hbm.at[wid], idx_all)

        def fire_gather(j, bf):
            pltpu.async_copy(tab_hbm.at[idx_all.at[j]], rows[bf], gsem[bf])

        def wait_gather(j, bf):
            pltpu.make_async_copy(
                tab_hbm.at[idx_all.at[j]], rows[bf], gsem[bf]
            ).wait()

        def fire_scatters(j, bf):
            for i in range(_RPC):
                pltpu.async_copy(
                    rows[bf].at[pl.ds(i * s, s)],
                    out_hbm.at[base + j * _RPC + i],
                    ssem[bf],
                )

        def wait_scatters(j, bf):
            for i in range(_RPC):
                pltpu.make_async_copy(
                    rows[bf].at[pl.ds(i * s, s)],
                    out_hbm.at[base + j * _RPC + i],
                    ssem[bf],
                ).wait()

        for bf in range(_NBUF):
            fire_gather(bf, bf)

        def outer(i, carry):
            for bf in range(_NBUF):
                j = i * _NBUF + bf
                wait_gather(j, bf)
                fire_scatters(j, bf)
                wait_scatters(j, bf)
                fire_gather(j + _NBUF, bf)
            return carry

        lax.fori_loop(0, n_chunks // _NBUF - 1, outer, 0)

        for bf in range(_NBUF):
            j = n_chunks - _NBUF + bf
            wait_gather(j, bf)
            fire_scatters(j, bf)
        for bf in range(_NBUF):
            j = n_chunks - _NBUF + bf
            wait_scatters(j, bf)

    return k(x3d, embedding)


def kernel(X, embedding):
    b, s = X.shape
    info = plsc.get_sparse_core_info()
    nw = info.num_cores * info.num_subcores
    rpw = b // nw
    assert b == nw * rpw and rpw % (_RPC * _NBUF) == 0 and _RPC * s <= 128
    x3d = X.reshape(nw, rpw // _RPC, _RPC * s).astype(jnp.int32)
    return _sc_gather(x3d, embedding, b, s)
